# baseline (device time: 28933 ns/iter reference)
import jax
import jax.numpy as jnp
from jax import lax
from jax.experimental import pallas as pl
from jax.experimental.pallas import tpu as pltpu

N_DEV = 16


def kernel(q, k, v):
    s_per, d = q.shape

    def body(q_ref, k_ref, v_ref, out_ref, big_send, big_recv, send_sem, recv_sem):
        my = lax.axis_index("i")
        s = lax.rem(my, 4)
        partner = my - s + jnp.where(lax.rem(s, 2) == 0, s + 1, s - 1)
        for i in range(16):
            big_send[i, 0] = k_ref[...].astype(jnp.bfloat16)
            big_send[i, 1] = v_ref[...].astype(jnp.bfloat16)
        barrier = pltpu.get_barrier_semaphore()
        for off in range(1, N_DEV):
            peer = lax.rem(my + off, N_DEV)
            pl.semaphore_signal(barrier, inc=1, device_id=(peer,),
                                device_id_type=pl.DeviceIdType.MESH)
        pl.semaphore_wait(barrier, N_DEV - 1)
        rdma = pltpu.make_async_remote_copy(
            src_ref=big_send, dst_ref=big_recv,
            send_sem=send_sem, recv_sem=recv_sem,
            device_id=(partner,), device_id_type=pl.DeviceIdType.MESH)
        rdma.start()
        rdma.wait()
        out_ref[...] = q_ref[...] + big_recv[0, 0].astype(jnp.float32) + big_recv[15, 1].astype(jnp.float32)

    return pl.pallas_call(
        body,
        out_shape=jax.ShapeDtypeStruct((s_per, d), jnp.float32),
        in_specs=[pl.BlockSpec(memory_space=pltpu.VMEM)] * 3,
        out_specs=pl.BlockSpec(memory_space=pltpu.VMEM),
        scratch_shapes=[
            pltpu.VMEM((16, 2, s_per, d), jnp.bfloat16),
            pltpu.VMEM((16, 2, s_per, d), jnp.bfloat16),
            pltpu.SemaphoreType.DMA,
            pltpu.SemaphoreType.DMA,
        ],
        compiler_params=pltpu.CompilerParams(collective_id=0),
    )(q, k, v)


# device time: 16733 ns/iter; 1.7291x vs baseline; 1.7291x over previous
import jax
import jax.numpy as jnp
from jax import lax
from jax.experimental import pallas as pl
from jax.experimental.pallas import tpu as pltpu

N_DEV = 16
N_Z = 4
N_S = 4


def kernel(q, k, v):
    s_per, d = q.shape
    scale = 1.0 / (d**0.5)

    def body(
        q_ref,
        k_ref,
        v_ref,
        out_ref,
        qbuf,
        kvbuf,
        vpad,
        psend_buf,
        precv_buf,
        qsend_sems,
        qrecv_sems,
        kvsend_sems,
        kvrecv_sems,
        psend_sems,
        precv_sems,
    ):
        my = lax.axis_index("i")
        z = my // N_S
        s = lax.rem(my, N_S)

        def col_peer(dz):
            return lax.rem(z + dz, N_Z) * N_S + s

        def plane_peer(ds):
            return z * N_S + lax.rem(s + ds, N_S)

        qbuf[z] = (q_ref[...] * scale).astype(jnp.bfloat16)
        kvbuf[s, 0] = k_ref[...].astype(jnp.bfloat16)
        kvbuf[s, 1] = v_ref[...].astype(jnp.bfloat16)
        vpad[...] = jnp.zeros((N_S, s_per, 2 * d), jnp.bfloat16)
        ones_col = jnp.ones((s_per, 1), jnp.bfloat16)

        def fill_vpad(si):
            vpad[si, :, :d] = kvbuf[si, 1]
            vpad[si, :, d : d + 1] = ones_col

        fill_vpad(s)

        barrier = pltpu.get_barrier_semaphore()
        for dz in range(1, N_Z):
            pl.semaphore_signal(
                barrier,
                inc=1,
                device_id=(col_peer(dz),),
                device_id_type=pl.DeviceIdType.MESH,
            )
        for ds in range(1, N_S):
            pl.semaphore_signal(
                barrier,
                inc=1,
                device_id=(plane_peer(ds),),
                device_id_type=pl.DeviceIdType.MESH,
            )
        pl.semaphore_wait(barrier, 6)

        sends = []

        for dz in range(1, N_Z):
            rdma = pltpu.make_async_remote_copy(
                src_ref=qbuf.at[z],
                dst_ref=qbuf.at[z],
                send_sem=qsend_sems.at[dz - 1],
                recv_sem=qrecv_sems.at[z],
                device_id=(col_peer(dz),),
                device_id_type=pl.DeviceIdType.MESH,
            )
            rdma.start()
            sends.append(rdma)
        for ds in (2, 1, 3):
            rdma = pltpu.make_async_remote_copy(
                src_ref=kvbuf.at[s],
                dst_ref=kvbuf.at[s],
                send_sem=kvsend_sems.at[ds - 1],
                recv_sem=kvrecv_sems.at[s],
                device_id=(plane_peer(ds),),
                device_id_type=pl.DeviceIdType.MESH,
            )
            rdma.start()
            sends.append(rdma)

        accs = [
            jnp.zeros((s_per, 2 * d), dtype=jnp.float32) for _ in range(N_Z)
        ]

        def pair_update(dz, si):
            zi = lax.rem(z + dz, N_Z)
            sc = lax.dot_general(
                qbuf[zi],
                kvbuf[si, 0],
                (((1,), (1,)), ((), ())),
                preferred_element_type=jnp.float32,
            )
            p = jnp.exp(sc)
            accs[dz] = accs[dz] + lax.dot(
                p.astype(jnp.bfloat16),
                vpad[si],
                preferred_element_type=jnp.float32,
            )

        def wait(dst, sem):
            recv = pltpu.make_async_remote_copy(
                src_ref=dst,
                dst_ref=dst,
                send_sem=qsend_sems.at[0],
                recv_sem=sem,
                device_id=(my,),
                device_id_type=pl.DeviceIdType.MESH,
            )
            recv.wait_recv()

        pair_update(0, s)
        for dz in range(1, N_Z):
            wait(qbuf.at[lax.rem(z + dz, N_Z)], qrecv_sems.at[lax.rem(z + dz, N_Z)])
            pair_update(dz, s)
        for ds in range(1, N_S):
            si = lax.rem(s + ds, N_S)
            wait(kvbuf.at[si], kvrecv_sems.at[si])
            fill_vpad(si)
            for dz in range(N_Z):
                pair_update(dz, si)

        for dz in range(1, N_Z):
            psend_buf[dz - 1] = accs[dz].astype(jnp.bfloat16)
            rdma = pltpu.make_async_remote_copy(
                src_ref=psend_buf.at[dz - 1],
                dst_ref=precv_buf.at[z],
                send_sem=psend_sems.at[dz - 1],
                recv_sem=precv_sems.at[z],
                device_id=(col_peer(dz),),
                device_id_type=pl.DeviceIdType.MESH,
            )
            rdma.start()
            sends.append(rdma)

        acc_tot = accs[0]
        for dz in range(1, N_Z):
            zp = lax.rem(z + dz, N_Z)
            wait(precv_buf.at[zp], precv_sems.at[zp])
            acc_tot = acc_tot + precv_buf[zp].astype(jnp.float32)

        for rdma in sends:
            rdma.wait_send()

        out_ref[...] = acc_tot[:, :d] / acc_tot[:, d : d + 1]

    return pl.pallas_call(
        body,
        out_shape=jax.ShapeDtypeStruct((s_per, d), jnp.float32),
        in_specs=[pl.BlockSpec(memory_space=pltpu.VMEM)] * 3,
        out_specs=pl.BlockSpec(memory_space=pltpu.VMEM),
        scratch_shapes=[
            pltpu.VMEM((N_Z, s_per, d), jnp.bfloat16),
            pltpu.VMEM((N_S, 2, s_per, d), jnp.bfloat16),
            pltpu.VMEM((N_S, s_per, 2 * d), jnp.bfloat16),
            pltpu.VMEM((N_Z - 1, s_per, 2 * d), jnp.bfloat16),
            pltpu.VMEM((N_Z, s_per, 2 * d), jnp.bfloat16),
            pltpu.SemaphoreType.DMA((N_Z - 1,)),
            pltpu.SemaphoreType.DMA((N_Z,)),
            pltpu.SemaphoreType.DMA((N_S - 1,)),
            pltpu.SemaphoreType.DMA((N_S,)),
            pltpu.SemaphoreType.DMA((N_Z - 1,)),
            pltpu.SemaphoreType.DMA((N_Z,)),
        ],
        compiler_params=pltpu.CompilerParams(collective_id=0),
    )(q, k, v)
